# baseline (speedup 1.0000x reference)
import functools

import jax
import jax.numpy as jnp
from jax.experimental import pallas as pl
from jax.experimental.pallas import tpu as pltpu
from jax.experimental.pallas import tpu_sc as plsc

V = 1000000
D = 16
B = 4096
L = 200
H = 20
P = 8
BB = 256
C = B // BB
G = BB // P
RPC = L * BB // P
CONST_MIN = -4294967295.0
W_GATHER = 128


def _transpose_tc(hist_idx):
    def body(x_ref, o_ref):
        o_ref[0] = x_ref[...].T

    return pl.pallas_call(
        body,
        grid=(C,),
        in_specs=[pl.BlockSpec((BB, L), lambda i: (i, 0))],
        out_specs=pl.BlockSpec((1, L, BB), lambda i: (i, 0, 0)),
        out_shape=jax.ShapeDtypeStruct((C, L, BB), jnp.int32),
        compiler_params=pltpu.CompilerParams(
            dimension_semantics=("arbitrary",)),
    )(hist_idx)


def _gather_sc(emb_table, ci, hist_t):
    nq = ci.shape[1]
    nk = C * L * BB
    par = BB // W_GATHER
    mesh = plsc.VectorSubcoreMesh(core_axis_name="core",
                                  subcore_axis_name="subcore")

    @functools.partial(
        pl.kernel,
        out_type=(jax.ShapeDtypeStruct((nq, D), jnp.float32),
                  jax.ShapeDtypeStruct((nk, D), jnp.float32)),
        mesh=mesh,
        compiler_params=pltpu.CompilerParams(use_tc_tiling_on_sc=False),
    )
    def gk(emb_hbm, ci_hbm, hi_hbm, q_hbm, k_hbm):
        def body(i_vmem, o_vmem):
            pltpu.sync_copy(emb_hbm.at[i_vmem.at[0]], o_vmem)

        def body3(i_vmem, o_vmem):
            pltpu.sync_copy(emb_hbm.at[i_vmem.at[0, 0]], o_vmem)

        pltpu.emit_pipeline(
            body,
            grid=(nq // W_GATHER,),
            in_specs=[pl.BlockSpec((1, W_GATHER), lambda i: (0, i))],
            out_specs=[pl.BlockSpec((W_GATHER, D), lambda i: (i, 0))],
            core_axis_name=("core", "subcore"),
            dimension_semantics=(pltpu.PARALLEL,),
        )(ci_hbm, q_hbm)
        pltpu.emit_pipeline(
            body3,
            grid=(C, L, par),
            in_specs=[pl.BlockSpec((1, 1, W_GATHER),
                                   lambda c, l, h: (c, l, h))],
            out_specs=[pl.BlockSpec((W_GATHER, D),
                                    lambda c, l, h: (c * L * par + l * par + h,
                                                     0))],
            core_axis_name=("core", "subcore"),
            dimension_semantics=(pltpu.PARALLEL,) * 3,
        )(hi_hbm, k_hbm)

    return gk(emb_table, ci, hist_t)


def _tc_body(qp_ref, kp_ref, hid_ref, W1x_ref, Ax_ref, W2x_ref, Wfx_ref,
             b1x_ref, b2x_ref, Ex_ref, out_ref):
    kp = kp_ref[0]
    qp = qp_ref[0]
    qA = jnp.dot(qp, Ax_ref[...],
                 preferred_element_type=jnp.float32) + b1x_ref[...]
    qAt = jnp.broadcast_to(qA[None], (L, G, P * H)).reshape(RPC, P * H)
    qpt = jnp.broadcast_to(qp[None], (L, G, P * D)).reshape(RPC, P * D)
    X1 = jnp.concatenate([kp, kp * qpt], axis=1)
    h1 = jax.nn.sigmoid(
        jnp.dot(X1, W1x_ref[...], preferred_element_type=jnp.float32) + qAt)
    h2 = jax.nn.sigmoid(
        jnp.dot(h1, W2x_ref[...], preferred_element_type=jnp.float32)
        + b2x_ref[...])
    logits = jnp.dot(h2, Wfx_ref[...],
                     preferred_element_type=jnp.float32)
    mask = hid_ref[0] != 0
    logits = jnp.where(mask, logits, jnp.float32(CONST_MIN))
    lg = logits.reshape(L, G, P)
    m = jnp.max(lg, axis=0)
    e = jnp.exp(lg - m[None])
    s = jnp.sum(e, axis=0)
    att = (e / s[None]).reshape(RPC, P)
    attw = jnp.dot(att, Ex_ref[...],
                   preferred_element_type=jnp.float32)
    acc = (attw * kp).reshape(L, G, P * D)
    out_ref[0] = jnp.sum(acc, axis=0)


def _blockdiag(M, n):
    r, c = M.shape
    out = jnp.zeros((n * r, n * c), M.dtype)
    for p in range(n):
        out = jax.lax.dynamic_update_slice(out, M, (p * r, p * c))
    return out


def _attention_tc(qv, kv, hv, W1x, Ax, W2x, Wfx, b1x, b2x, Ex,
                  interpret=False):
    full = lambda shape: pl.BlockSpec(shape, lambda i: tuple(0 for _ in shape))
    return pl.pallas_call(
        _tc_body,
        grid=(C,),
        in_specs=[
            pl.BlockSpec((1, G, P * D), lambda i: (i, 0, 0)),
            pl.BlockSpec((1, RPC, P * D), lambda i: (i, 0, 0)),
            pl.BlockSpec((1, RPC, P), lambda i: (i, 0, 0)),
            full((2 * P * D, P * H)),
            full((P * D, P * H)),
            full((P * H, P * H)),
            full((P * H, P)),
            full((1, P * H)),
            full((1, P * H)),
            full((P, P * D)),
        ],
        out_specs=pl.BlockSpec((1, G, P * D), lambda i: (i, 0, 0)),
        out_shape=jax.ShapeDtypeStruct((C, G, P * D), jnp.float32),
        compiler_params=pltpu.CompilerParams(
            dimension_semantics=("arbitrary",)),
        interpret=interpret,
    )(qv, kv, hv, W1x, Ax, W2x, Wfx, b1x, b2x, Ex)


def _pack_weights(W1, b1, W2, b2, Wf):
    W1a, W1b, W1c, W1d = W1[0:D], W1[D:2 * D], W1[2 * D:3 * D], W1[3 * D:4 * D]
    A = W1a + W1c
    Bm = W1b - W1c
    Cm = W1d
    W1x = jnp.concatenate([_blockdiag(Bm, P), _blockdiag(Cm, P)], axis=0)
    Ax = _blockdiag(A, P)
    W2x = _blockdiag(W2, P)
    Wfx = _blockdiag(Wf, P)
    b1x = jnp.tile(b1, P).reshape(1, P * H)
    b2x = jnp.tile(b2, P).reshape(1, P * H)
    Ex = _blockdiag(jnp.ones((1, D), jnp.float32), P)
    return W1x, Ax, W2x, Wfx, b1x, b2x, Ex


def kernel(cand_idx, hist_idx, emb_table, W1, b1, W2, b2, Wf, bf):
    ci = cand_idx.astype(jnp.int32).reshape(1, B)
    hist_perm = _transpose_tc(hist_idx.astype(jnp.int32))
    q_rows, k_rows = _gather_sc(emb_table, ci, hist_perm)
    qv = q_rows.reshape(C, G, P * D)
    kv = k_rows.reshape(C, RPC, P * D)
    hv = hist_perm.reshape(C, RPC, P)
    packed = _pack_weights(W1, b1, W2, b2, Wf)
    out = _attention_tc(qv, kv, hv, *packed)
    return out.reshape(B, D)


# speedup vs baseline: 1.0352x; 1.0352x over previous
import functools

import jax
import jax.numpy as jnp
from jax.experimental import pallas as pl
from jax.experimental.pallas import tpu as pltpu
from jax.experimental.pallas import tpu_sc as plsc

V = 1000000
D = 16
B = 4096
L = 200
H = 20
P = 8
BB = 256
C = B // BB
G = BB // P
RPC = L * BB // P
CONST_MIN = -4294967295.0
W_GATHER = 128


def _transpose_tc(hist_idx):
    def body(x_ref, o_ref):
        o_ref[0] = x_ref[...].T

    return pl.pallas_call(
        body,
        grid=(C,),
        in_specs=[pl.BlockSpec((BB, L), lambda i: (i, 0))],
        out_specs=pl.BlockSpec((1, L, BB), lambda i: (i, 0, 0)),
        out_shape=jax.ShapeDtypeStruct((C, L, BB), jnp.int32),
        compiler_params=pltpu.CompilerParams(
            dimension_semantics=("arbitrary",)),
    )(hist_idx)


def _gather_sc(emb_table, ci, hist_t):
    nq = ci.shape[1]
    nk = C * L * BB
    par = BB // W_GATHER
    mesh = plsc.VectorSubcoreMesh(core_axis_name="core",
                                  subcore_axis_name="subcore")

    @functools.partial(
        pl.kernel,
        out_type=(jax.ShapeDtypeStruct((nq, D), jnp.float32),
                  jax.ShapeDtypeStruct((nk, D), jnp.float32)),
        mesh=mesh,
        compiler_params=pltpu.CompilerParams(use_tc_tiling_on_sc=False),
    )
    def gk(emb_hbm, ci_hbm, hi_hbm, q_hbm, k_hbm):
        def body(i_vmem, o_vmem):
            pltpu.sync_copy(emb_hbm.at[i_vmem.at[0]], o_vmem)

        def body3(i_vmem, o_vmem):
            pltpu.sync_copy(emb_hbm.at[i_vmem.at[0, 0]], o_vmem)

        pltpu.emit_pipeline(
            body,
            grid=(nq // W_GATHER,),
            in_specs=[pl.BlockSpec((1, W_GATHER), lambda i: (0, i))],
            out_specs=[pl.BlockSpec((W_GATHER, D), lambda i: (i, 0))],
            core_axis_name=("core", "subcore"),
            dimension_semantics=(pltpu.PARALLEL,),
        )(ci_hbm, q_hbm)
        pltpu.emit_pipeline(
            body3,
            grid=(C, L, par),
            in_specs=[pl.BlockSpec((1, 1, W_GATHER),
                                   lambda c, l, h: (c, l, h))],
            out_specs=[pl.BlockSpec((W_GATHER, D),
                                    lambda c, l, h: (c * L * par + l * par + h,
                                                     0))],
            core_axis_name=("core", "subcore"),
            dimension_semantics=(pltpu.PARALLEL,) * 3,
        )(hi_hbm, k_hbm)

    return gk(emb_table, ci, hist_t)


def _tc_body(qp_ref, kp_ref, W1x_ref, Ax_ref, W2x_ref, Wfx_ref,
             b1x_ref, b2x_ref, e0_ref, S_ref, out_ref):
    kp = kp_ref[0]
    qp = qp_ref[0]
    qA = jnp.dot(qp, Ax_ref[...],
                 preferred_element_type=jnp.float32) + b1x_ref[0:1]
    qAt = jnp.broadcast_to(qA[None], (L, G, P * H)).reshape(RPC, P * H)
    qpt = jnp.broadcast_to(qp[None], (L, G, P * D)).reshape(RPC, P * D)
    X1 = jnp.concatenate([kp, kp * qpt], axis=1)
    h1 = jax.nn.sigmoid(
        jnp.dot(X1, W1x_ref[...], preferred_element_type=jnp.float32) + qAt)
    h2 = jax.nn.sigmoid(
        jnp.dot(h1, W2x_ref[...], preferred_element_type=jnp.float32)
        + b2x_ref[0:1])
    lgx = jnp.dot(h2, Wfx_ref[...],
                  preferred_element_type=jnp.float32)
    d01 = (kp != e0_ref[0:1]).astype(jnp.float32)
    nz = jnp.dot(d01, S_ref[...],
                 preferred_element_type=jnp.float32)
    lgx = jnp.where(nz > 0.0, lgx, jnp.float32(CONST_MIN))
    lg = lgx.reshape(L, G, P * D)
    m = jnp.max(lg, axis=0)
    e = jnp.exp(lg - m[None])
    s = jnp.sum(e, axis=0)
    attw = (e / s[None]).reshape(RPC, P * D)
    acc = (attw * kp).reshape(L, G, P * D)
    out_ref[0] = jnp.sum(acc, axis=0)


def _blockdiag(M, n):
    r, c = M.shape
    out = jnp.zeros((n * r, n * c), M.dtype)
    for p in range(n):
        out = jax.lax.dynamic_update_slice(out, M, (p * r, p * c))
    return out


def _attention_tc(qv, kv, W1x, Ax, W2x, Wfx, b1x, b2x, e0, S,
                  interpret=False):
    full = lambda shape: pl.BlockSpec(shape, lambda i: tuple(0 for _ in shape))
    return pl.pallas_call(
        _tc_body,
        grid=(C,),
        in_specs=[
            pl.BlockSpec((1, G, P * D), lambda i: (i, 0, 0)),
            pl.BlockSpec((1, RPC, P * D), lambda i: (i, 0, 0)),
            full((2 * P * D, P * H)),
            full((P * D, P * H)),
            full((P * H, P * H)),
            full((P * H, P * D)),
            full((8, P * H)),
            full((8, P * H)),
            full((8, P * D)),
            full((P * D, P * D)),
        ],
        out_specs=pl.BlockSpec((1, G, P * D), lambda i: (i, 0, 0)),
        out_shape=jax.ShapeDtypeStruct((C, G, P * D), jnp.float32),
        compiler_params=pltpu.CompilerParams(
            dimension_semantics=("arbitrary",)),
        interpret=interpret,
    )(qv, kv, W1x, Ax, W2x, Wfx, b1x, b2x, e0, S)


def _pack_weights(W1, b1, W2, b2, Wf, emb0):
    W1a, W1b, W1c, W1d = W1[0:D], W1[D:2 * D], W1[2 * D:3 * D], W1[3 * D:4 * D]
    A = W1a + W1c
    Bm = W1b - W1c
    Cm = W1d
    W1x = jnp.concatenate([_blockdiag(Bm, P), _blockdiag(Cm, P)], axis=0)
    Ax = _blockdiag(A, P)
    W2x = _blockdiag(W2, P)
    Wfx = _blockdiag(jnp.tile(Wf, (1, D)), P)
    b1x = jnp.tile(jnp.tile(b1, P).reshape(1, P * H), (8, 1))
    b2x = jnp.tile(jnp.tile(b2, P).reshape(1, P * H), (8, 1))
    e0 = jnp.tile(jnp.tile(emb0, P).reshape(1, P * D), (8, 1))
    S = _blockdiag(jnp.ones((D, D), jnp.float32), P)
    return W1x, Ax, W2x, Wfx, b1x, b2x, e0, S


def kernel(cand_idx, hist_idx, emb_table, W1, b1, W2, b2, Wf, bf):
    ci = cand_idx.astype(jnp.int32).reshape(1, B)
    hist_perm = _transpose_tc(hist_idx.astype(jnp.int32))
    q_rows, k_rows = _gather_sc(emb_table, ci, hist_perm)
    qv = q_rows.reshape(C, G, P * D)
    kv = k_rows.reshape(C, RPC, P * D)
    packed = _pack_weights(W1, b1, W2, b2, Wf, emb_table[0])
    out = _attention_tc(qv, kv, *packed)
    return out.reshape(B, D)


# speedup vs baseline: 1.0879x; 1.0509x over previous
import functools

import jax
import jax.numpy as jnp
from jax.experimental import pallas as pl
from jax.experimental.pallas import tpu as pltpu
from jax.experimental.pallas import tpu_sc as plsc

V = 1000000
D = 16
B = 4096
L = 200
H = 20
P = 8
BB = 256
C = B // BB
G = BB // P
RPC = L * BB // P
CONST_MIN = -4294967295.0
W_GATHER = 128
PAR = BB // W_GATHER
NCH = 4


def _transpose_tc(hist_idx):
    def body(x_ref, o_ref):
        o_ref[0] = x_ref[...].T.reshape(L, PAR, W_GATHER)

    return pl.pallas_call(
        body,
        grid=(C,),
        in_specs=[pl.BlockSpec((BB, L), lambda i: (i, 0))],
        out_specs=pl.BlockSpec((1, L, PAR, W_GATHER), lambda i: (i, 0, 0, 0)),
        out_shape=jax.ShapeDtypeStruct((C, L, PAR, W_GATHER), jnp.int32),
        compiler_params=pltpu.CompilerParams(
            dimension_semantics=("arbitrary",)),
    )(hist_idx)


def _gather_q_sc(emb_table, ci):
    mesh = plsc.VectorSubcoreMesh(core_axis_name="core",
                                  subcore_axis_name="subcore")

    @functools.partial(
        pl.kernel,
        out_type=jax.ShapeDtypeStruct((B, D), jnp.float32),
        mesh=mesh,
        compiler_params=pltpu.CompilerParams(use_tc_tiling_on_sc=False),
    )
    def gk(emb_hbm, ci_hbm, q_hbm):
        def body(i_vmem, o_vmem):
            pltpu.sync_copy(emb_hbm.at[i_vmem.at[0]], o_vmem)

        pltpu.emit_pipeline(
            body,
            grid=(B // W_GATHER,),
            in_specs=[pl.BlockSpec((1, W_GATHER), lambda i: (i, 0))],
            out_specs=[pl.BlockSpec((W_GATHER, D), lambda i: (i, 0))],
            core_axis_name=("core", "subcore"),
            dimension_semantics=(pltpu.PARALLEL,),
        )(ci_hbm, q_hbm)

    return gk(emb_table, ci)


def _gather_k_sc(emb_table, hist_t, c0, nc):
    mesh = plsc.VectorSubcoreMesh(core_axis_name="core",
                                  subcore_axis_name="subcore")

    @functools.partial(
        pl.kernel,
        out_type=jax.ShapeDtypeStruct((nc * L * BB, D), jnp.float32),
        mesh=mesh,
        compiler_params=pltpu.CompilerParams(use_tc_tiling_on_sc=False),
    )
    def gk(emb_hbm, hi_hbm, k_hbm):
        def body(i_vmem, o_vmem):
            pltpu.sync_copy(emb_hbm.at[i_vmem.at[0, 0, 0]], o_vmem)

        pltpu.emit_pipeline(
            body,
            grid=(nc, L, PAR),
            in_specs=[pl.BlockSpec((1, 1, 1, W_GATHER),
                                   lambda c, l, h: (c0 + c, l, h, 0))],
            out_specs=[pl.BlockSpec((W_GATHER, D),
                                    lambda c, l, h: (c * L * PAR + l * PAR + h,
                                                     0))],
            core_axis_name=("core", "subcore"),
            dimension_semantics=(pltpu.PARALLEL,) * 3,
        )(hi_hbm, k_hbm)

    return gk(emb_table, hist_t)


def _tc_body(qp_ref, kp_ref, W1x_ref, Ax_ref, W2x_ref, Wfx_ref,
             b1x_ref, b2x_ref, e0_ref, S_ref, out_ref):
    kp = kp_ref[0]
    qp = qp_ref[0]
    qA = jnp.dot(qp, Ax_ref[...],
                 preferred_element_type=jnp.float32) + b1x_ref[0:1]
    qAt = jnp.broadcast_to(qA[None], (L, G, P * H)).reshape(RPC, P * H)
    qpt = jnp.broadcast_to(qp[None], (L, G, P * D)).reshape(RPC, P * D)
    X1 = jnp.concatenate([kp, kp * qpt], axis=1)
    h1 = jax.nn.sigmoid(
        jnp.dot(X1, W1x_ref[...], preferred_element_type=jnp.float32) + qAt)
    h2 = jax.nn.sigmoid(
        jnp.dot(h1, W2x_ref[...], preferred_element_type=jnp.float32)
        + b2x_ref[0:1])
    lgx = jnp.dot(h2, Wfx_ref[...],
                  preferred_element_type=jnp.float32)
    d01 = (kp != e0_ref[0:1]).astype(jnp.float32)
    nz = jnp.dot(d01, S_ref[...],
                 preferred_element_type=jnp.float32)
    lgx = jnp.where(nz > 0.0, lgx, jnp.float32(CONST_MIN))
    lg = lgx.reshape(L, G, P * D)
    m = jnp.max(lg, axis=0)
    e = jnp.exp(lg - m[None])
    s = jnp.sum(e, axis=0)
    attw = (e / s[None]).reshape(RPC, P * D)
    acc = (attw * kp).reshape(L, G, P * D)
    out_ref[0] = jnp.sum(acc, axis=0)


def _blockdiag(M, n):
    r, c = M.shape
    out = jnp.zeros((n * r, n * c), M.dtype)
    for p in range(n):
        out = jax.lax.dynamic_update_slice(out, M, (p * r, p * c))
    return out


def _attention_tc(qv, kv, W1x, Ax, W2x, Wfx, b1x, b2x, e0, S, c0, nc,
                  interpret=False):
    full = lambda shape: pl.BlockSpec(shape, lambda i: tuple(0 for _ in shape))
    return pl.pallas_call(
        _tc_body,
        grid=(nc,),
        in_specs=[
            pl.BlockSpec((1, G, P * D), lambda i: (c0 + i, 0, 0)),
            pl.BlockSpec((1, RPC, P * D), lambda i: (i, 0, 0)),
            full((2 * P * D, P * H)),
            full((P * D, P * H)),
            full((P * H, P * H)),
            full((P * H, P * D)),
            full((8, P * H)),
            full((8, P * H)),
            full((8, P * D)),
            full((P * D, P * D)),
        ],
        out_specs=pl.BlockSpec((1, G, P * D), lambda i: (i, 0, 0)),
        out_shape=jax.ShapeDtypeStruct((nc, G, P * D), jnp.float32),
        compiler_params=pltpu.CompilerParams(
            dimension_semantics=("arbitrary",)),
        interpret=interpret,
    )(qv, kv, W1x, Ax, W2x, Wfx, b1x, b2x, e0, S)


def _pack_weights(W1, b1, W2, b2, Wf, emb0):
    W1a, W1b, W1c, W1d = W1[0:D], W1[D:2 * D], W1[2 * D:3 * D], W1[3 * D:4 * D]
    A = W1a + W1c
    Bm = W1b - W1c
    Cm = W1d
    W1x = jnp.concatenate([_blockdiag(Bm, P), _blockdiag(Cm, P)], axis=0)
    Ax = _blockdiag(A, P)
    W2x = _blockdiag(W2, P)
    Wfx = _blockdiag(jnp.tile(Wf, (1, D)), P)
    b1x = jnp.tile(jnp.tile(b1, P).reshape(1, P * H), (8, 1))
    b2x = jnp.tile(jnp.tile(b2, P).reshape(1, P * H), (8, 1))
    e0 = jnp.tile(jnp.tile(emb0, P).reshape(1, P * D), (8, 1))
    S = _blockdiag(jnp.ones((D, D), jnp.float32), P)
    return W1x, Ax, W2x, Wfx, b1x, b2x, e0, S


def kernel(cand_idx, hist_idx, emb_table, W1, b1, W2, b2, Wf, bf):
    ci = cand_idx.astype(jnp.int32).reshape(G, P * D)
    hist_perm = _transpose_tc(hist_idx.astype(jnp.int32))
    q_rows = _gather_q_sc(emb_table, ci)
    qv = q_rows.reshape(C, G, P * D)
    packed = _pack_weights(W1, b1, W2, b2, Wf, emb_table[0])
    nc = C // NCH
    outs = []
    for part in range(NCH):
        k_rows = _gather_k_sc(emb_table, hist_perm, part * nc, nc)
        kv = k_rows.reshape(nc, RPC, P * D)
        outs.append(_attention_tc(qv, kv, *packed, part * nc, nc))
    return jnp.concatenate(outs, axis=0).reshape(B, D)
